# baseline (speedup 1.0000x reference)
import jax
import jax.numpy as jnp
from jax.experimental import pallas as pl
from jax.experimental.pallas import tpu as pltpu

NUM_Q_HEADS = 32
NUM_KV_HEADS = 8
HEAD_DIM = 128
GQA = NUM_Q_HEADS // NUM_KV_HEADS
SCALE = HEAD_DIM ** -0.5
NUM_BLOCKS = 2048
BLOCK_SIZE = 16
B = 16
MAX_BLOCKS_PER_SEQ = 128
MAX_LEN = MAX_BLOCKS_PER_SEQ * BLOCK_SIZE
FUSED = NUM_KV_HEADS * HEAD_DIM
CHUNK = 32
CHUNK_TOK = CHUNK * BLOCK_SIZE
MAX_CHUNKS = MAX_BLOCKS_PER_SEQ // CHUNK
NBUF = 3
AHEAD = NBUF - 1
NEG_INF = float("-inf")


def _attn_body(nb_ref, bt_ref, sl_ref,
               qbd_ref, k_new_ref, v_new_ref, mask_ref, excl_ref, cnt_ref,
               kc_hbm, vc_hbm,
               out_ref,
               m_ref, l_ref, acc_ref, kcat_ref, vcat_ref, sem_ref):
    b = pl.program_id(0)
    nb = nb_ref[b]
    nchunks = (nb + CHUNK - 1) // CHUNK

    def _issue(c):
        slot = jax.lax.rem(c, NBUF)
        for j in range(CHUNK):
            pos = jnp.minimum(c * CHUNK + j, nb - 1)
            pb = bt_ref[b, pos]
            pltpu.make_async_copy(
                kc_hbm.at[pb],
                kcat_ref.at[slot, pl.ds(j * BLOCK_SIZE, BLOCK_SIZE)],
                sem_ref.at[slot, j]).start()
            pltpu.make_async_copy(
                vc_hbm.at[pb],
                vcat_ref.at[slot, pl.ds(j * BLOCK_SIZE, BLOCK_SIZE)],
                sem_ref.at[slot, j]).start()

    m_ref[...] = jnp.full_like(m_ref, NEG_INF)
    l_ref[...] = jnp.zeros_like(l_ref)
    acc_ref[...] = jnp.zeros_like(acc_ref)

    jax.lax.fori_loop(0, jnp.minimum(AHEAD, nchunks),
                      lambda c, _: (_issue(c), 0)[1], 0)

    def _chunk_body(c, _):
        @pl.when(c + AHEAD < nchunks)
        def _issue_ahead():
            _issue(c + AHEAD)

        slot = jax.lax.rem(c, NBUF)
        for j in range(CHUNK):
            pltpu.make_async_copy(
                kc_hbm.at[bt_ref[b, 0]],
                kcat_ref.at[slot, pl.ds(j * BLOCK_SIZE, BLOCK_SIZE)],
                sem_ref.at[slot, j]).wait()
            pltpu.make_async_copy(
                vc_hbm.at[bt_ref[b, 0]],
                vcat_ref.at[slot, pl.ds(j * BLOCK_SIZE, BLOCK_SIZE)],
                sem_ref.at[slot, j]).wait()

        kc = kcat_ref[slot]
        vc = vcat_ref[slot]
        s = jax.lax.dot_general(
            qbd_ref[0], kc, (((1,), (1,)), ((), ())),
            preferred_element_type=jnp.float32) * SCALE
        rem = sl_ref[b] - c * CHUNK_TOK
        lane = jax.lax.broadcasted_iota(jnp.int32, (NUM_Q_HEADS, CHUNK_TOK), 1)
        ex = excl_ref[0, 0, pl.ds(c * CHUNK_TOK, CHUNK_TOK)]
        keep = jnp.logical_and(lane < rem, (ex < 0.5)[None, :])
        s = jnp.where(keep, s, NEG_INF)
        m_old = m_ref[...]
        m_new = jnp.maximum(m_old, jnp.max(s, axis=1, keepdims=True))
        alpha = jnp.exp(m_old - m_new)
        p = jnp.exp(s - m_new)
        l_ref[...] = alpha * l_ref[...] + jnp.sum(p, axis=1, keepdims=True)
        pv = jax.lax.dot_general(
            p, vc, (((1,), (0,)), ((), ())),
            preferred_element_type=jnp.float32)
        acc_ref[...] = alpha * acc_ref[...] + pv
        m_ref[...] = m_new
        return 0

    jax.lax.fori_loop(0, nchunks, _chunk_body, 0)

    cnt = cnt_ref[0, 0]
    s_f = jax.lax.dot_general(
        qbd_ref[0], k_new_ref[...], (((1,), (1,)), ((), ())),
        preferred_element_type=jnp.float32) * SCALE
    s_f = jnp.where((cnt > 0.5)[None, :], s_f, NEG_INF)
    m_old = m_ref[...]
    m_new = jnp.maximum(m_old, jnp.max(s_f, axis=1, keepdims=True))
    alpha = jnp.exp(m_old - m_new)
    p_f = jnp.exp(s_f - m_new) * cnt[None, :]
    l_fin = alpha * l_ref[...] + jnp.sum(p_f, axis=1, keepdims=True)
    pv_f = jax.lax.dot_general(
        p_f, v_new_ref[...], (((1,), (0,)), ((), ())),
        preferred_element_type=jnp.float32)
    acc_fin = alpha * acc_ref[...] + pv_f

    a = acc_fin * mask_ref[...]
    o = a[:, 0:HEAD_DIM]
    for j in range(1, NUM_KV_HEADS):
        o = o + a[:, j * HEAD_DIM : (j + 1) * HEAD_DIM]
    out_ref[0] = o / l_fin


@jax.jit
def _paged_attn(q, k, v, k_cache, v_cache, slot_mapping, block_tables,
                seq_lens):
    nb = (seq_lens + BLOCK_SIZE - 1) // BLOCK_SIZE
    kc3 = k_cache.reshape(NUM_BLOCKS, BLOCK_SIZE, FUSED)
    vc3 = v_cache.reshape(NUM_BLOCKS, BLOCK_SIZE, FUSED)
    k2 = k.reshape(B, FUSED)
    v2 = v.reshape(B, FUSED)
    bd = (jnp.arange(FUSED)[None, :] // HEAD_DIM
          == jnp.arange(NUM_Q_HEADS)[:, None] // GQA)
    bd = bd.astype(jnp.float32)
    q_bd = jnp.tile(q, (1, 1, NUM_KV_HEADS)) * bd[None]

    slot_i32 = slot_mapping.astype(jnp.int32)
    slot_blk = slot_i32 // BLOCK_SIZE
    slot_off = slot_i32 % BLOCK_SIZE
    occ = block_tables[:, :, None] == slot_blk[None, None, :]
    hit_off = (slot_off[None, None, :, None]
               == jnp.arange(BLOCK_SIZE)[None, None, None, :])
    excl = jnp.any(occ[:, :, :, None] & hit_off, axis=2)
    excl = excl.reshape(B, 1, MAX_LEN).astype(jnp.float32)
    pos_tok = (jnp.arange(MAX_BLOCKS_PER_SEQ)[None, :, None] * BLOCK_SIZE
               + slot_off[None, None, :])
    in_range = pos_tok < seq_lens[:, None, None]
    cnt = jnp.sum((occ & in_range).astype(jnp.float32), axis=1)
    wi = jnp.arange(B)
    dup_later = jnp.any((slot_i32[None, :] == slot_i32[:, None])
                        & (wi[None, :] > wi[:, None]), axis=1)
    cnt = cnt * (~dup_later)[None, :].astype(jnp.float32)
    cnt = cnt.reshape(B, 1, B)

    grid_spec = pltpu.PrefetchScalarGridSpec(
        num_scalar_prefetch=3,
        grid=(B,),
        in_specs=[
            pl.BlockSpec((1, NUM_Q_HEADS, FUSED), lambda b, *_: (b, 0, 0)),
            pl.BlockSpec((B, FUSED), lambda b, *_: (0, 0)),
            pl.BlockSpec((B, FUSED), lambda b, *_: (0, 0)),
            pl.BlockSpec((NUM_Q_HEADS, FUSED), lambda b, *_: (0, 0)),
            pl.BlockSpec((1, 1, MAX_LEN), lambda b, *_: (b, 0, 0)),
            pl.BlockSpec((1, 1, B), lambda b, *_: (b, 0, 0)),
            pl.BlockSpec(memory_space=pltpu.MemorySpace.HBM),
            pl.BlockSpec(memory_space=pltpu.MemorySpace.HBM),
        ],
        out_specs=pl.BlockSpec((1, NUM_Q_HEADS, HEAD_DIM),
                               lambda b, *_: (b, 0, 0)),
        scratch_shapes=[
            pltpu.VMEM((NUM_Q_HEADS, 1), jnp.float32),
            pltpu.VMEM((NUM_Q_HEADS, 1), jnp.float32),
            pltpu.VMEM((NUM_Q_HEADS, FUSED), jnp.float32),
            pltpu.VMEM((NBUF, CHUNK_TOK, FUSED), jnp.float32),
            pltpu.VMEM((NBUF, CHUNK_TOK, FUSED), jnp.float32),
            pltpu.SemaphoreType.DMA((NBUF, CHUNK)),
        ],
    )
    return pl.pallas_call(
        _attn_body,
        grid_spec=grid_spec,
        out_shape=jax.ShapeDtypeStruct((B, NUM_Q_HEADS, HEAD_DIM),
                                       jnp.float32),
        compiler_params=pltpu.CompilerParams(
            dimension_semantics=("parallel",)),
    )(nb, block_tables, seq_lens,
      q_bd, k2, v2, bd, excl, cnt, kc3, vc3)


def kernel(q, k, v, k_cache, v_cache, slot_mapping, block_tables, seq_lens,
           query_lens, is_prefill):
    del query_lens, is_prefill
    return _paged_attn(q, k, v, k_cache, v_cache, slot_mapping, block_tables,
                       seq_lens)


# speedup vs baseline: 1.0012x; 1.0012x over previous
import jax
import jax.numpy as jnp
from jax.experimental import pallas as pl
from jax.experimental.pallas import tpu as pltpu

NUM_Q_HEADS = 32
NUM_KV_HEADS = 8
HEAD_DIM = 128
GQA = NUM_Q_HEADS // NUM_KV_HEADS
SCALE = HEAD_DIM ** -0.5
NUM_BLOCKS = 2048
BLOCK_SIZE = 16
B = 16
MAX_BLOCKS_PER_SEQ = 128
MAX_LEN = MAX_BLOCKS_PER_SEQ * BLOCK_SIZE
FUSED = NUM_KV_HEADS * HEAD_DIM
CHUNK = 32
CHUNK_TOK = CHUNK * BLOCK_SIZE
MAX_CHUNKS = MAX_BLOCKS_PER_SEQ // CHUNK
NSPLIT = 4
SPB = CHUNK // NSPLIT
SPT = SPB * BLOCK_SIZE
NBUF = 3
AHEAD = NBUF - 1
NEG_INF = float("-inf")


def _attn_body(nb_ref, bt_ref, sl_ref,
               qbd_ref, k_new_ref, v_new_ref, mask_ref, excl_ref, cnt_ref,
               kc_hbm, vc_hbm,
               out_ref,
               *rest):
    m_ref, l_ref, acc_ref = rest[0:3]
    kcat_refs = rest[3:3 + NSPLIT]
    vcat_refs = rest[3 + NSPLIT:3 + 2 * NSPLIT]
    sem_ref = rest[3 + 2 * NSPLIT]

    b = pl.program_id(0)
    nb = nb_ref[b]
    nchunks = (nb + CHUNK - 1) // CHUNK

    def _issue(c):
        slot = jax.lax.rem(c, NBUF)
        for j in range(CHUNK):
            pos = jnp.minimum(c * CHUNK + j, nb - 1)
            pb = bt_ref[b, pos]
            q_, r_ = divmod(j, SPB)
            pltpu.make_async_copy(
                kc_hbm.at[pb],
                kcat_refs[q_].at[slot, pl.ds(r_ * BLOCK_SIZE, BLOCK_SIZE)],
                sem_ref.at[slot, j]).start()
            pltpu.make_async_copy(
                vc_hbm.at[pb],
                vcat_refs[q_].at[slot, pl.ds(r_ * BLOCK_SIZE, BLOCK_SIZE)],
                sem_ref.at[slot, j]).start()

    m_ref[...] = jnp.full_like(m_ref, NEG_INF)
    l_ref[...] = jnp.zeros_like(l_ref)
    acc_ref[...] = jnp.zeros_like(acc_ref)

    jax.lax.fori_loop(0, jnp.minimum(AHEAD, nchunks),
                      lambda c, _: (_issue(c), 0)[1], 0)

    def _chunk_body(c, _):
        @pl.when(c + AHEAD < nchunks)
        def _issue_ahead():
            _issue(c + AHEAD)

        slot = jax.lax.rem(c, NBUF)
        for j in range(CHUNK):
            q_, r_ = divmod(j, SPB)
            pltpu.make_async_copy(
                kc_hbm.at[bt_ref[b, 0]],
                kcat_refs[q_].at[slot, pl.ds(r_ * BLOCK_SIZE, BLOCK_SIZE)],
                sem_ref.at[slot, j]).wait()
            pltpu.make_async_copy(
                vc_hbm.at[bt_ref[b, 0]],
                vcat_refs[q_].at[slot, pl.ds(r_ * BLOCK_SIZE, BLOCK_SIZE)],
                sem_ref.at[slot, j]).wait()

        s = jnp.concatenate(
            [jax.lax.dot_general(
                qbd_ref[0], kcat_refs[q_][slot], (((1,), (1,)), ((), ())),
                preferred_element_type=jnp.float32)
             for q_ in range(NSPLIT)], axis=1) * SCALE
        rem = sl_ref[b] - c * CHUNK_TOK
        lane = jax.lax.broadcasted_iota(jnp.int32, (NUM_Q_HEADS, CHUNK_TOK), 1)
        ex = excl_ref[0, 0, pl.ds(c * CHUNK_TOK, CHUNK_TOK)]
        keep = jnp.logical_and(lane < rem, (ex < 0.5)[None, :])
        s = jnp.where(keep, s, NEG_INF)
        m_old = m_ref[...]
        m_new = jnp.maximum(m_old, jnp.max(s, axis=1, keepdims=True))
        alpha = jnp.exp(m_old - m_new)
        p = jnp.exp(s - m_new)
        l_ref[...] = alpha * l_ref[...] + jnp.sum(p, axis=1, keepdims=True)
        pv = jax.lax.dot_general(
            p[:, 0:SPT], vcat_refs[0][slot], (((1,), (0,)), ((), ())),
            preferred_element_type=jnp.float32)
        for q_ in range(1, NSPLIT):
            pv = pv + jax.lax.dot_general(
                p[:, q_ * SPT:(q_ + 1) * SPT], vcat_refs[q_][slot],
                (((1,), (0,)), ((), ())),
                preferred_element_type=jnp.float32)
        acc_ref[...] = alpha * acc_ref[...] + pv
        m_ref[...] = m_new
        return 0

    jax.lax.fori_loop(0, nchunks, _chunk_body, 0)

    cnt = cnt_ref[0, 0]
    s_f = jax.lax.dot_general(
        qbd_ref[0], k_new_ref[...], (((1,), (1,)), ((), ())),
        preferred_element_type=jnp.float32) * SCALE
    s_f = jnp.where((cnt > 0.5)[None, :], s_f, NEG_INF)
    m_old = m_ref[...]
    m_new = jnp.maximum(m_old, jnp.max(s_f, axis=1, keepdims=True))
    alpha = jnp.exp(m_old - m_new)
    p_f = jnp.exp(s_f - m_new) * cnt[None, :]
    l_fin = alpha * l_ref[...] + jnp.sum(p_f, axis=1, keepdims=True)
    pv_f = jax.lax.dot_general(
        p_f, v_new_ref[...], (((1,), (0,)), ((), ())),
        preferred_element_type=jnp.float32)
    acc_fin = alpha * acc_ref[...] + pv_f

    a = acc_fin * mask_ref[...]
    o = a[:, 0:HEAD_DIM]
    for j in range(1, NUM_KV_HEADS):
        o = o + a[:, j * HEAD_DIM : (j + 1) * HEAD_DIM]
    out_ref[0] = o / l_fin


@jax.jit
def _paged_attn(q, k, v, k_cache, v_cache, slot_mapping, block_tables,
                seq_lens):
    nb = (seq_lens + BLOCK_SIZE - 1) // BLOCK_SIZE
    kc3 = k_cache.reshape(NUM_BLOCKS, BLOCK_SIZE, FUSED)
    vc3 = v_cache.reshape(NUM_BLOCKS, BLOCK_SIZE, FUSED)
    k2 = k.reshape(B, FUSED)
    v2 = v.reshape(B, FUSED)
    bd = (jnp.arange(FUSED)[None, :] // HEAD_DIM
          == jnp.arange(NUM_Q_HEADS)[:, None] // GQA)
    bd = bd.astype(jnp.float32)
    q_bd = jnp.tile(q, (1, 1, NUM_KV_HEADS)) * bd[None]

    slot_i32 = slot_mapping.astype(jnp.int32)
    slot_blk = slot_i32 // BLOCK_SIZE
    slot_off = slot_i32 % BLOCK_SIZE
    occ = block_tables[:, :, None] == slot_blk[None, None, :]
    hit_off = (slot_off[None, None, :, None]
               == jnp.arange(BLOCK_SIZE)[None, None, None, :])
    excl = jnp.any(occ[:, :, :, None] & hit_off, axis=2)
    excl = excl.reshape(B, 1, MAX_LEN).astype(jnp.float32)
    pos_tok = (jnp.arange(MAX_BLOCKS_PER_SEQ)[None, :, None] * BLOCK_SIZE
               + slot_off[None, None, :])
    in_range = pos_tok < seq_lens[:, None, None]
    cnt = jnp.sum((occ & in_range).astype(jnp.float32), axis=1)
    wi = jnp.arange(B)
    dup_later = jnp.any((slot_i32[None, :] == slot_i32[:, None])
                        & (wi[None, :] > wi[:, None]), axis=1)
    cnt = cnt * (~dup_later)[None, :].astype(jnp.float32)
    cnt = cnt.reshape(B, 1, B)

    grid_spec = pltpu.PrefetchScalarGridSpec(
        num_scalar_prefetch=3,
        grid=(B,),
        in_specs=[
            pl.BlockSpec((1, NUM_Q_HEADS, FUSED), lambda b, *_: (b, 0, 0)),
            pl.BlockSpec((B, FUSED), lambda b, *_: (0, 0)),
            pl.BlockSpec((B, FUSED), lambda b, *_: (0, 0)),
            pl.BlockSpec((NUM_Q_HEADS, FUSED), lambda b, *_: (0, 0)),
            pl.BlockSpec((1, 1, MAX_LEN), lambda b, *_: (b, 0, 0)),
            pl.BlockSpec((1, 1, B), lambda b, *_: (b, 0, 0)),
            pl.BlockSpec(memory_space=pltpu.MemorySpace.HBM),
            pl.BlockSpec(memory_space=pltpu.MemorySpace.HBM),
        ],
        out_specs=pl.BlockSpec((1, NUM_Q_HEADS, HEAD_DIM),
                               lambda b, *_: (b, 0, 0)),
        scratch_shapes=(
            [pltpu.VMEM((NUM_Q_HEADS, 1), jnp.float32),
             pltpu.VMEM((NUM_Q_HEADS, 1), jnp.float32),
             pltpu.VMEM((NUM_Q_HEADS, FUSED), jnp.float32)]
            + [pltpu.VMEM((NBUF, SPT, FUSED), jnp.float32)
               for _ in range(2 * NSPLIT)]
            + [pltpu.SemaphoreType.DMA((NBUF, CHUNK))]
        ),
    )
    return pl.pallas_call(
        _attn_body,
        grid_spec=grid_spec,
        out_shape=jax.ShapeDtypeStruct((B, NUM_Q_HEADS, HEAD_DIM),
                                       jnp.float32),
        compiler_params=pltpu.CompilerParams(
            dimension_semantics=("arbitrary",)),
    )(nb, block_tables, seq_lens,
      q_bd, k2, v2, bd, excl, cnt, kc3, vc3)


def kernel(q, k, v, k_cache, v_cache, slot_mapping, block_tables, seq_lens,
           query_lens, is_prefill):
    del query_lens, is_prefill
    return _paged_attn(q, k, v, k_cache, v_cache, slot_mapping, block_tables,
                       seq_lens)


# speedup vs baseline: 1.0030x; 1.0017x over previous
import jax
import jax.numpy as jnp
from jax.experimental import pallas as pl
from jax.experimental.pallas import tpu as pltpu

NUM_Q_HEADS = 32
NUM_KV_HEADS = 8
HEAD_DIM = 128
GQA = NUM_Q_HEADS // NUM_KV_HEADS
SCALE = HEAD_DIM ** -0.5
NUM_BLOCKS = 2048
BLOCK_SIZE = 16
B = 16
MAX_BLOCKS_PER_SEQ = 128
MAX_LEN = MAX_BLOCKS_PER_SEQ * BLOCK_SIZE
FUSED = NUM_KV_HEADS * HEAD_DIM
CHUNK = 32
CHUNK_TOK = CHUNK * BLOCK_SIZE
MAX_CHUNKS = MAX_BLOCKS_PER_SEQ // CHUNK
NSPLIT = 4
SPB = CHUNK // NSPLIT
SPT = SPB * BLOCK_SIZE
NBUF = 3
AHEAD = NBUF - 1
NEG_INF = float("-inf")


def _attn_body(nb_ref, bt_ref, sl_ref,
               qbd_ref, k_new_ref, v_new_ref, mask_ref, excl_ref, cnt_ref,
               kc_hbm_a, kc_hbm_b, vc_hbm_a, vc_hbm_b,
               out_ref,
               *rest):
    m_ref, l_ref, acc_ref = rest[0:3]
    kcat_refs = rest[3:3 + NSPLIT]
    vcat_refs = rest[3 + NSPLIT:3 + 2 * NSPLIT]
    sem_ref = rest[3 + 2 * NSPLIT]

    b = pl.program_id(0)
    nb = nb_ref[b]
    nchunks = (nb + CHUNK - 1) // CHUNK

    def _issue(c):
        slot = jax.lax.rem(c, NBUF)
        for j in range(CHUNK):
            pos = jnp.minimum(c * CHUNK + j, nb - 1)
            pb = bt_ref[b, pos]
            q_, r_ = divmod(j, SPB)
            kc_hbm = kc_hbm_a if j % 2 == 0 else kc_hbm_b
            vc_hbm = vc_hbm_a if j % 2 == 0 else vc_hbm_b
            pltpu.make_async_copy(
                kc_hbm.at[pb],
                kcat_refs[q_].at[slot, pl.ds(r_ * BLOCK_SIZE, BLOCK_SIZE)],
                sem_ref.at[slot, j]).start()
            pltpu.make_async_copy(
                vc_hbm.at[pb],
                vcat_refs[q_].at[slot, pl.ds(r_ * BLOCK_SIZE, BLOCK_SIZE)],
                sem_ref.at[slot, j]).start()

    m_ref[...] = jnp.full_like(m_ref, NEG_INF)
    l_ref[...] = jnp.zeros_like(l_ref)
    acc_ref[...] = jnp.zeros_like(acc_ref)

    jax.lax.fori_loop(0, jnp.minimum(AHEAD, nchunks),
                      lambda c, _: (_issue(c), 0)[1], 0)

    def _chunk_body(c, _):
        @pl.when(c + AHEAD < nchunks)
        def _issue_ahead():
            _issue(c + AHEAD)

        slot = jax.lax.rem(c, NBUF)
        for j in range(CHUNK):
            q_, r_ = divmod(j, SPB)
            kc_hbm = kc_hbm_a if j % 2 == 0 else kc_hbm_b
            vc_hbm = vc_hbm_a if j % 2 == 0 else vc_hbm_b
            pltpu.make_async_copy(
                kc_hbm.at[bt_ref[b, 0]],
                kcat_refs[q_].at[slot, pl.ds(r_ * BLOCK_SIZE, BLOCK_SIZE)],
                sem_ref.at[slot, j]).wait()
            pltpu.make_async_copy(
                vc_hbm.at[bt_ref[b, 0]],
                vcat_refs[q_].at[slot, pl.ds(r_ * BLOCK_SIZE, BLOCK_SIZE)],
                sem_ref.at[slot, j]).wait()

        s = jnp.concatenate(
            [jax.lax.dot_general(
                qbd_ref[0], kcat_refs[q_][slot], (((1,), (1,)), ((), ())),
                preferred_element_type=jnp.float32)
             for q_ in range(NSPLIT)], axis=1) * SCALE
        rem = sl_ref[b] - c * CHUNK_TOK
        lane = jax.lax.broadcasted_iota(jnp.int32, (NUM_Q_HEADS, CHUNK_TOK), 1)
        ex = excl_ref[0, 0, pl.ds(c * CHUNK_TOK, CHUNK_TOK)]
        keep = jnp.logical_and(lane < rem, (ex < 0.5)[None, :])
        s = jnp.where(keep, s, NEG_INF)
        m_old = m_ref[...]
        m_new = jnp.maximum(m_old, jnp.max(s, axis=1, keepdims=True))
        alpha = jnp.exp(m_old - m_new)
        p = jnp.exp(s - m_new)
        l_ref[...] = alpha * l_ref[...] + jnp.sum(p, axis=1, keepdims=True)
        pv = jax.lax.dot_general(
            p[:, 0:SPT], vcat_refs[0][slot], (((1,), (0,)), ((), ())),
            preferred_element_type=jnp.float32)
        for q_ in range(1, NSPLIT):
            pv = pv + jax.lax.dot_general(
                p[:, q_ * SPT:(q_ + 1) * SPT], vcat_refs[q_][slot],
                (((1,), (0,)), ((), ())),
                preferred_element_type=jnp.float32)
        acc_ref[...] = alpha * acc_ref[...] + pv
        m_ref[...] = m_new
        return 0

    jax.lax.fori_loop(0, nchunks, _chunk_body, 0)

    cnt = cnt_ref[0, 0]
    s_f = jax.lax.dot_general(
        qbd_ref[0], k_new_ref[...], (((1,), (1,)), ((), ())),
        preferred_element_type=jnp.float32) * SCALE
    s_f = jnp.where((cnt > 0.5)[None, :], s_f, NEG_INF)
    m_old = m_ref[...]
    m_new = jnp.maximum(m_old, jnp.max(s_f, axis=1, keepdims=True))
    alpha = jnp.exp(m_old - m_new)
    p_f = jnp.exp(s_f - m_new) * cnt[None, :]
    l_fin = alpha * l_ref[...] + jnp.sum(p_f, axis=1, keepdims=True)
    pv_f = jax.lax.dot_general(
        p_f, v_new_ref[...], (((1,), (0,)), ((), ())),
        preferred_element_type=jnp.float32)
    acc_fin = alpha * acc_ref[...] + pv_f

    a = acc_fin * mask_ref[...]
    o = a[:, 0:HEAD_DIM]
    for j in range(1, NUM_KV_HEADS):
        o = o + a[:, j * HEAD_DIM : (j + 1) * HEAD_DIM]
    out_ref[0] = o / l_fin


@jax.jit
def _paged_attn(q, k, v, k_cache, v_cache, slot_mapping, block_tables,
                seq_lens):
    nb = (seq_lens + BLOCK_SIZE - 1) // BLOCK_SIZE
    kc3 = k_cache.reshape(NUM_BLOCKS, BLOCK_SIZE, FUSED)
    vc3 = v_cache.reshape(NUM_BLOCKS, BLOCK_SIZE, FUSED)
    k2 = k.reshape(B, FUSED)
    v2 = v.reshape(B, FUSED)
    bd = (jnp.arange(FUSED)[None, :] // HEAD_DIM
          == jnp.arange(NUM_Q_HEADS)[:, None] // GQA)
    bd = bd.astype(jnp.float32)
    q_bd = jnp.tile(q, (1, 1, NUM_KV_HEADS)) * bd[None]

    slot_i32 = slot_mapping.astype(jnp.int32)
    slot_blk = slot_i32 // BLOCK_SIZE
    slot_off = slot_i32 % BLOCK_SIZE
    occ = block_tables[:, :, None] == slot_blk[None, None, :]
    hit_off = (slot_off[None, None, :, None]
               == jnp.arange(BLOCK_SIZE)[None, None, None, :])
    excl = jnp.any(occ[:, :, :, None] & hit_off, axis=2)
    excl = excl.reshape(B, 1, MAX_LEN).astype(jnp.float32)
    pos_tok = (jnp.arange(MAX_BLOCKS_PER_SEQ)[None, :, None] * BLOCK_SIZE
               + slot_off[None, None, :])
    in_range = pos_tok < seq_lens[:, None, None]
    cnt = jnp.sum((occ & in_range).astype(jnp.float32), axis=1)
    wi = jnp.arange(B)
    dup_later = jnp.any((slot_i32[None, :] == slot_i32[:, None])
                        & (wi[None, :] > wi[:, None]), axis=1)
    cnt = cnt * (~dup_later)[None, :].astype(jnp.float32)
    cnt = cnt.reshape(B, 1, B)

    grid_spec = pltpu.PrefetchScalarGridSpec(
        num_scalar_prefetch=3,
        grid=(B,),
        in_specs=[
            pl.BlockSpec((1, NUM_Q_HEADS, FUSED), lambda b, *_: (b, 0, 0)),
            pl.BlockSpec((B, FUSED), lambda b, *_: (0, 0)),
            pl.BlockSpec((B, FUSED), lambda b, *_: (0, 0)),
            pl.BlockSpec((NUM_Q_HEADS, FUSED), lambda b, *_: (0, 0)),
            pl.BlockSpec((1, 1, MAX_LEN), lambda b, *_: (b, 0, 0)),
            pl.BlockSpec((1, 1, B), lambda b, *_: (b, 0, 0)),
            pl.BlockSpec(memory_space=pltpu.MemorySpace.HBM),
            pl.BlockSpec(memory_space=pltpu.MemorySpace.HBM),
            pl.BlockSpec(memory_space=pltpu.MemorySpace.HBM),
            pl.BlockSpec(memory_space=pltpu.MemorySpace.HBM),
        ],
        out_specs=pl.BlockSpec((1, NUM_Q_HEADS, HEAD_DIM),
                               lambda b, *_: (b, 0, 0)),
        scratch_shapes=(
            [pltpu.VMEM((NUM_Q_HEADS, 1), jnp.float32),
             pltpu.VMEM((NUM_Q_HEADS, 1), jnp.float32),
             pltpu.VMEM((NUM_Q_HEADS, FUSED), jnp.float32)]
            + [pltpu.VMEM((NBUF, SPT, FUSED), jnp.float32)
               for _ in range(2 * NSPLIT)]
            + [pltpu.SemaphoreType.DMA((NBUF, CHUNK))]
        ),
    )
    return pl.pallas_call(
        _attn_body,
        grid_spec=grid_spec,
        out_shape=jax.ShapeDtypeStruct((B, NUM_Q_HEADS, HEAD_DIM),
                                       jnp.float32),
        compiler_params=pltpu.CompilerParams(
            dimension_semantics=("arbitrary",)),
    )(nb, block_tables, seq_lens,
      q_bd, k2, v2, bd, excl, cnt, kc3, kc3, vc3, vc3)


def kernel(q, k, v, k_cache, v_cache, slot_mapping, block_tables, seq_lens,
           query_lens, is_prefill):
    del query_lens, is_prefill
    return _paged_attn(q, k, v, k_cache, v_cache, slot_mapping, block_tables,
                       seq_lens)


# speedup vs baseline: 1.0239x; 1.0208x over previous
import jax
import jax.numpy as jnp
from jax.experimental import pallas as pl
from jax.experimental.pallas import tpu as pltpu

NUM_Q_HEADS = 32
NUM_KV_HEADS = 8
HEAD_DIM = 128
GQA = NUM_Q_HEADS // NUM_KV_HEADS
SCALE = HEAD_DIM ** -0.5
NUM_BLOCKS = 2048
BLOCK_SIZE = 16
B = 16
MAX_BLOCKS_PER_SEQ = 128
MAX_LEN = MAX_BLOCKS_PER_SEQ * BLOCK_SIZE
FUSED = NUM_KV_HEADS * HEAD_DIM
CHUNK = 16
CHUNK_TOK = CHUNK * BLOCK_SIZE
MAX_CHUNKS = MAX_BLOCKS_PER_SEQ // CHUNK
MAX_TOTAL = B * MAX_CHUNKS
NBUF = 4
AHEAD = NBUF - 1
NEG_INF = float("-inf")


def _attn_body(gseq_ref, gloc_ref, glast_ref, tot_ref, nb_ref, bt_ref,
               sl_ref,
               qbd_ref, k_new_ref, v_new_ref, mask_ref, excl_ref, cnt_ref,
               kc_hbm, vc_hbm,
               out_ref,
               m_ref, l_ref, acc_ref, kcat_ref, vcat_ref, sem_ref):
    g = pl.program_id(0)
    tot = tot_ref[0]

    def _issue(gi):
        s_ = gseq_ref[gi]
        c_ = gloc_ref[gi]
        nbs = nb_ref[s_]
        slot = jax.lax.rem(gi, NBUF)
        for j in range(CHUNK):
            @pl.when(c_ * CHUNK + j < nbs)
            def _start():
                pb = bt_ref[s_, c_ * CHUNK + j]
                pltpu.make_async_copy(
                    kc_hbm.at[pb],
                    kcat_ref.at[slot, pl.ds(j * BLOCK_SIZE, BLOCK_SIZE)],
                    sem_ref.at[slot, j]).start()
                pltpu.make_async_copy(
                    vc_hbm.at[pb],
                    vcat_ref.at[slot, pl.ds(j * BLOCK_SIZE, BLOCK_SIZE)],
                    sem_ref.at[slot, j]).start()

    @pl.when(g == 0)
    def _prologue():
        vcat_ref[...] = jnp.zeros_like(vcat_ref)
        jax.lax.fori_loop(0, jnp.minimum(AHEAD, tot),
                          lambda gi, _: (_issue(gi), 0)[1], 0)

    @pl.when(g + AHEAD < tot)
    def _issue_ahead():
        _issue(g + AHEAD)

    @pl.when(g < tot)
    def _work():
        seq = gseq_ref[g]
        c = gloc_ref[g]
        nbs = nb_ref[seq]
        slot = jax.lax.rem(g, NBUF)

        @pl.when(c == 0)
        def _init():
            m_ref[...] = jnp.full_like(m_ref, NEG_INF)
            l_ref[...] = jnp.zeros_like(l_ref)
            acc_ref[...] = jnp.zeros_like(acc_ref)

        for j in range(CHUNK):
            @pl.when(c * CHUNK + j < nbs)
            def _wait():
                pltpu.make_async_copy(
                    kc_hbm.at[bt_ref[seq, 0]],
                    kcat_ref.at[slot, pl.ds(j * BLOCK_SIZE, BLOCK_SIZE)],
                    sem_ref.at[slot, j]).wait()
                pltpu.make_async_copy(
                    vc_hbm.at[bt_ref[seq, 0]],
                    vcat_ref.at[slot, pl.ds(j * BLOCK_SIZE, BLOCK_SIZE)],
                    sem_ref.at[slot, j]).wait()

        kc = kcat_ref[slot]
        vc = vcat_ref[slot]
        s = jax.lax.dot_general(
            qbd_ref[0], kc, (((1,), (1,)), ((), ())),
            preferred_element_type=jnp.float32) * SCALE
        rem = sl_ref[seq] - c * CHUNK_TOK
        lane = jax.lax.broadcasted_iota(jnp.int32, (NUM_Q_HEADS, CHUNK_TOK), 1)
        ex = excl_ref[0, 0, pl.ds(c * CHUNK_TOK, CHUNK_TOK)]
        keep = jnp.logical_and(lane < rem, (ex < 0.5)[None, :])
        s = jnp.where(keep, s, NEG_INF)
        m_old = m_ref[...]
        m_new = jnp.maximum(m_old, jnp.max(s, axis=1, keepdims=True))
        alpha = jnp.exp(m_old - m_new)
        p = jnp.exp(s - m_new)
        l_ref[...] = alpha * l_ref[...] + jnp.sum(p, axis=1, keepdims=True)
        pv = jax.lax.dot_general(
            p, vc, (((1,), (0,)), ((), ())),
            preferred_element_type=jnp.float32)
        acc_ref[...] = alpha * acc_ref[...] + pv
        m_ref[...] = m_new

        @pl.when(glast_ref[g] == 1)
        def _finalize():
            cnt = cnt_ref[0, 0]
            s_f = jax.lax.dot_general(
                qbd_ref[0], k_new_ref[...], (((1,), (1,)), ((), ())),
                preferred_element_type=jnp.float32) * SCALE
            s_f = jnp.where((cnt > 0.5)[None, :], s_f, NEG_INF)
            mo = m_ref[...]
            mn = jnp.maximum(mo, jnp.max(s_f, axis=1, keepdims=True))
            al = jnp.exp(mo - mn)
            p_f = jnp.exp(s_f - mn) * cnt[None, :]
            l_fin = al * l_ref[...] + jnp.sum(p_f, axis=1, keepdims=True)
            pv_f = jax.lax.dot_general(
                p_f, v_new_ref[...], (((1,), (0,)), ((), ())),
                preferred_element_type=jnp.float32)
            acc_fin = al * acc_ref[...] + pv_f

            a = acc_fin * mask_ref[...]
            o = a[:, 0:HEAD_DIM]
            for h in range(1, NUM_KV_HEADS):
                o = o + a[:, h * HEAD_DIM : (h + 1) * HEAD_DIM]
            out_ref[0] = o / l_fin


@jax.jit
def _paged_attn(q, k, v, k_cache, v_cache, slot_mapping, block_tables,
                seq_lens):
    nb = ((seq_lens + BLOCK_SIZE - 1) // BLOCK_SIZE).astype(jnp.int32)
    kc3 = k_cache.reshape(NUM_BLOCKS, BLOCK_SIZE, FUSED)
    vc3 = v_cache.reshape(NUM_BLOCKS, BLOCK_SIZE, FUSED)
    k2 = k.reshape(B, FUSED)
    v2 = v.reshape(B, FUSED)
    bd = (jnp.arange(FUSED)[None, :] // HEAD_DIM
          == jnp.arange(NUM_Q_HEADS)[:, None] // GQA)
    bd = bd.astype(jnp.float32)
    q_bd = jnp.tile(q, (1, 1, NUM_KV_HEADS)) * bd[None]

    nchunks = ((nb + CHUNK - 1) // CHUNK).astype(jnp.int32)
    cum = jnp.concatenate([jnp.zeros((1,), jnp.int32),
                           jnp.cumsum(nchunks).astype(jnp.int32)])
    tot = cum[B].reshape(1)
    gi = jnp.arange(MAX_TOTAL, dtype=jnp.int32)
    gseq = jnp.minimum(
        jnp.sum((gi[:, None] >= cum[None, 1:]).astype(jnp.int32), axis=1),
        B - 1).astype(jnp.int32)
    gloc = gi - cum[gseq]
    glast = (gloc == nchunks[gseq] - 1).astype(jnp.int32)

    slot_i32 = slot_mapping.astype(jnp.int32)
    slot_blk = slot_i32 // BLOCK_SIZE
    slot_off = slot_i32 % BLOCK_SIZE
    occ = block_tables[:, :, None] == slot_blk[None, None, :]
    hit_off = (slot_off[None, None, :, None]
               == jnp.arange(BLOCK_SIZE)[None, None, None, :])
    excl = jnp.any(occ[:, :, :, None] & hit_off, axis=2)
    excl = excl.reshape(B, 1, MAX_LEN).astype(jnp.float32)
    pos_tok = (jnp.arange(MAX_BLOCKS_PER_SEQ)[None, :, None] * BLOCK_SIZE
               + slot_off[None, None, :])
    in_range = pos_tok < seq_lens[:, None, None]
    cnt = jnp.sum((occ & in_range).astype(jnp.float32), axis=1)
    wi = jnp.arange(B)
    dup_later = jnp.any((slot_i32[None, :] == slot_i32[:, None])
                        & (wi[None, :] > wi[:, None]), axis=1)
    cnt = cnt * (~dup_later)[None, :].astype(jnp.float32)
    cnt = cnt.reshape(B, 1, B)

    grid_spec = pltpu.PrefetchScalarGridSpec(
        num_scalar_prefetch=7,
        grid=(MAX_TOTAL,),
        in_specs=[
            pl.BlockSpec((1, NUM_Q_HEADS, FUSED),
                         lambda g, gseq, *_: (gseq[g], 0, 0)),
            pl.BlockSpec((B, FUSED), lambda g, *_: (0, 0)),
            pl.BlockSpec((B, FUSED), lambda g, *_: (0, 0)),
            pl.BlockSpec((NUM_Q_HEADS, FUSED), lambda g, *_: (0, 0)),
            pl.BlockSpec((1, 1, MAX_LEN),
                         lambda g, gseq, *_: (gseq[g], 0, 0)),
            pl.BlockSpec((1, 1, B),
                         lambda g, gseq, *_: (gseq[g], 0, 0)),
            pl.BlockSpec(memory_space=pltpu.MemorySpace.HBM),
            pl.BlockSpec(memory_space=pltpu.MemorySpace.HBM),
        ],
        out_specs=pl.BlockSpec((1, NUM_Q_HEADS, HEAD_DIM),
                               lambda g, gseq, *_: (gseq[g], 0, 0)),
        scratch_shapes=[
            pltpu.VMEM((NUM_Q_HEADS, 1), jnp.float32),
            pltpu.VMEM((NUM_Q_HEADS, 1), jnp.float32),
            pltpu.VMEM((NUM_Q_HEADS, FUSED), jnp.float32),
            pltpu.VMEM((NBUF, CHUNK_TOK, FUSED), jnp.float32),
            pltpu.VMEM((NBUF, CHUNK_TOK, FUSED), jnp.float32),
            pltpu.SemaphoreType.DMA((NBUF, CHUNK)),
        ],
    )
    return pl.pallas_call(
        _attn_body,
        grid_spec=grid_spec,
        out_shape=jax.ShapeDtypeStruct((B, NUM_Q_HEADS, HEAD_DIM),
                                       jnp.float32),
        compiler_params=pltpu.CompilerParams(
            dimension_semantics=("arbitrary",)),
    )(gseq, gloc, glast, tot, nb, block_tables, seq_lens,
      q_bd, k2, v2, bd, excl, cnt, kc3, vc3)


def kernel(q, k, v, k_cache, v_cache, slot_mapping, block_tables, seq_lens,
           query_lens, is_prefill):
    del query_lens, is_prefill
    return _paged_attn(q, k, v, k_cache, v_cache, slot_mapping, block_tables,
                       seq_lens)


# speedup vs baseline: 1.0521x; 1.0276x over previous
import jax
import jax.numpy as jnp
from jax.experimental import pallas as pl
from jax.experimental.pallas import tpu as pltpu

NUM_Q_HEADS = 32
NUM_KV_HEADS = 8
HEAD_DIM = 128
GQA = NUM_Q_HEADS // NUM_KV_HEADS
SCALE = HEAD_DIM ** -0.5
NUM_BLOCKS = 2048
BLOCK_SIZE = 16
B = 16
MAX_BLOCKS_PER_SEQ = 128
MAX_LEN = MAX_BLOCKS_PER_SEQ * BLOCK_SIZE
FUSED = NUM_KV_HEADS * HEAD_DIM
CHUNK = 16
CHUNK_TOK = CHUNK * BLOCK_SIZE
MAX_CHUNKS = MAX_BLOCKS_PER_SEQ // CHUNK
NBUF = 4
AHEAD = NBUF - 1
NEG_INF = float("-inf")


def _attn_body(nb_ref, bt_ref, sl_ref,
               qbd_ref, k_new_ref, v_new_ref, mask_ref, excl_ref, cnt_ref,
               kc_hbm, vc_hbm,
               out_ref,
               m_ref, l_ref, acc_ref, kcat_ref, vcat_ref, sem_ref):
    b = pl.program_id(0)
    nb = nb_ref[b]
    nchunks = (nb + CHUNK - 1) // CHUNK

    def _copies(c, j, slot):
        pb = bt_ref[b, c * CHUNK + j]
        kcp = pltpu.make_async_copy(
            kc_hbm.at[pb],
            kcat_ref.at[slot, pl.ds(j * BLOCK_SIZE, BLOCK_SIZE)],
            sem_ref.at[slot])
        vcp = pltpu.make_async_copy(
            vc_hbm.at[pb],
            vcat_ref.at[slot, pl.ds(j * BLOCK_SIZE, BLOCK_SIZE)],
            sem_ref.at[slot])
        return kcp, vcp

    def _nactive(c):
        return jnp.clip(nb - c * CHUNK, 0, CHUNK)

    def _issue(c):
        slot = jax.lax.rem(c, NBUF)

        def _one(j, _):
            kcp, vcp = _copies(c, j, slot)
            kcp.start()
            vcp.start()
            return 0

        jax.lax.fori_loop(0, _nactive(c), _one, 0)

    @pl.when(b == 0)
    def _init_vcat():
        vcat_ref[...] = jnp.zeros_like(vcat_ref)

    m_ref[...] = jnp.full_like(m_ref, NEG_INF)
    l_ref[...] = jnp.zeros_like(l_ref)
    acc_ref[...] = jnp.zeros_like(acc_ref)

    jax.lax.fori_loop(0, jnp.minimum(AHEAD, nchunks),
                      lambda c, _: (_issue(c), 0)[1], 0)

    def _chunk_body(c, _):
        @pl.when(c + AHEAD < nchunks)
        def _issue_ahead():
            _issue(c + AHEAD)

        slot = jax.lax.rem(c, NBUF)

        def _one_wait(j, _):
            kcp, vcp = _copies(c, j, slot)
            kcp.wait()
            vcp.wait()
            return 0

        jax.lax.fori_loop(0, _nactive(c), _one_wait, 0)

        kc = kcat_ref[slot]
        vc = vcat_ref[slot]
        s = jax.lax.dot_general(
            qbd_ref[0], kc, (((1,), (1,)), ((), ())),
            preferred_element_type=jnp.float32) * SCALE
        rem = sl_ref[b] - c * CHUNK_TOK
        lane = jax.lax.broadcasted_iota(jnp.int32, (NUM_Q_HEADS, CHUNK_TOK), 1)
        ex = excl_ref[0, 0, pl.ds(c * CHUNK_TOK, CHUNK_TOK)]
        keep = jnp.logical_and(lane < rem, (ex < 0.5)[None, :])
        s = jnp.where(keep, s, NEG_INF)
        m_old = m_ref[...]
        m_new = jnp.maximum(m_old, jnp.max(s, axis=1, keepdims=True))
        alpha = jnp.exp(m_old - m_new)
        p = jnp.exp(s - m_new)
        l_ref[...] = alpha * l_ref[...] + jnp.sum(p, axis=1, keepdims=True)
        pv = jax.lax.dot_general(
            p, vc, (((1,), (0,)), ((), ())),
            preferred_element_type=jnp.float32)
        acc_ref[...] = alpha * acc_ref[...] + pv
        m_ref[...] = m_new
        return 0

    jax.lax.fori_loop(0, nchunks, _chunk_body, 0)

    cnt = cnt_ref[0, 0]
    s_f = jax.lax.dot_general(
        qbd_ref[0], k_new_ref[...], (((1,), (1,)), ((), ())),
        preferred_element_type=jnp.float32) * SCALE
    s_f = jnp.where((cnt > 0.5)[None, :], s_f, NEG_INF)
    m_old = m_ref[...]
    m_new = jnp.maximum(m_old, jnp.max(s_f, axis=1, keepdims=True))
    alpha = jnp.exp(m_old - m_new)
    p_f = jnp.exp(s_f - m_new) * cnt[None, :]
    l_fin = alpha * l_ref[...] + jnp.sum(p_f, axis=1, keepdims=True)
    pv_f = jax.lax.dot_general(
        p_f, v_new_ref[...], (((1,), (0,)), ((), ())),
        preferred_element_type=jnp.float32)
    acc_fin = alpha * acc_ref[...] + pv_f

    a = acc_fin * mask_ref[...]
    o = a[:, 0:HEAD_DIM]
    for j in range(1, NUM_KV_HEADS):
        o = o + a[:, j * HEAD_DIM : (j + 1) * HEAD_DIM]
    out_ref[0] = o / l_fin


@jax.jit
def _paged_attn(q, k, v, k_cache, v_cache, slot_mapping, block_tables,
                seq_lens):
    nb = ((seq_lens + BLOCK_SIZE - 1) // BLOCK_SIZE).astype(jnp.int32)
    kc3 = k_cache.reshape(NUM_BLOCKS, BLOCK_SIZE, FUSED)
    vc3 = v_cache.reshape(NUM_BLOCKS, BLOCK_SIZE, FUSED)
    k2 = k.reshape(B, FUSED)
    v2 = v.reshape(B, FUSED)
    bd = (jnp.arange(FUSED)[None, :] // HEAD_DIM
          == jnp.arange(NUM_Q_HEADS)[:, None] // GQA)
    bd = bd.astype(jnp.float32)
    q_bd = jnp.tile(q, (1, 1, NUM_KV_HEADS)) * bd[None]

    slot_i32 = slot_mapping.astype(jnp.int32)
    slot_blk = slot_i32 // BLOCK_SIZE
    slot_off = slot_i32 % BLOCK_SIZE
    occ = block_tables[:, :, None] == slot_blk[None, None, :]
    hit_off = (slot_off[None, None, :, None]
               == jnp.arange(BLOCK_SIZE)[None, None, None, :])
    excl = jnp.any(occ[:, :, :, None] & hit_off, axis=2)
    excl = excl.reshape(B, 1, MAX_LEN).astype(jnp.float32)
    pos_tok = (jnp.arange(MAX_BLOCKS_PER_SEQ)[None, :, None] * BLOCK_SIZE
               + slot_off[None, None, :])
    in_range = pos_tok < seq_lens[:, None, None]
    cnt = jnp.sum((occ & in_range).astype(jnp.float32), axis=1)
    wi = jnp.arange(B)
    dup_later = jnp.any((slot_i32[None, :] == slot_i32[:, None])
                        & (wi[None, :] > wi[:, None]), axis=1)
    cnt = cnt * (~dup_later)[None, :].astype(jnp.float32)
    cnt = cnt.reshape(B, 1, B)

    grid_spec = pltpu.PrefetchScalarGridSpec(
        num_scalar_prefetch=3,
        grid=(B,),
        in_specs=[
            pl.BlockSpec((1, NUM_Q_HEADS, FUSED), lambda b, *_: (b, 0, 0)),
            pl.BlockSpec((B, FUSED), lambda b, *_: (0, 0)),
            pl.BlockSpec((B, FUSED), lambda b, *_: (0, 0)),
            pl.BlockSpec((NUM_Q_HEADS, FUSED), lambda b, *_: (0, 0)),
            pl.BlockSpec((1, 1, MAX_LEN), lambda b, *_: (b, 0, 0)),
            pl.BlockSpec((1, 1, B), lambda b, *_: (b, 0, 0)),
            pl.BlockSpec(memory_space=pltpu.MemorySpace.HBM),
            pl.BlockSpec(memory_space=pltpu.MemorySpace.HBM),
        ],
        out_specs=pl.BlockSpec((1, NUM_Q_HEADS, HEAD_DIM),
                               lambda b, *_: (b, 0, 0)),
        scratch_shapes=[
            pltpu.VMEM((NUM_Q_HEADS, 1), jnp.float32),
            pltpu.VMEM((NUM_Q_HEADS, 1), jnp.float32),
            pltpu.VMEM((NUM_Q_HEADS, FUSED), jnp.float32),
            pltpu.VMEM((NBUF, CHUNK_TOK, FUSED), jnp.float32),
            pltpu.VMEM((NBUF, CHUNK_TOK, FUSED), jnp.float32),
            pltpu.SemaphoreType.DMA((NBUF,)),
        ],
    )
    return pl.pallas_call(
        _attn_body,
        grid_spec=grid_spec,
        out_shape=jax.ShapeDtypeStruct((B, NUM_Q_HEADS, HEAD_DIM),
                                       jnp.float32),
        compiler_params=pltpu.CompilerParams(
            dimension_semantics=("arbitrary",)),
    )(nb, block_tables, seq_lens,
      q_bd, k2, v2, bd, excl, cnt, kc3, vc3)


def kernel(q, k, v, k_cache, v_cache, slot_mapping, block_tables, seq_lens,
           query_lens, is_prefill):
    del query_lens, is_prefill
    return _paged_attn(q, k, v, k_cache, v_cache, slot_mapping, block_tables,
                       seq_lens)
